# baseline (speedup 1.0000x reference)
import jax
import jax.numpy as jnp
from jax import lax
from jax.experimental import pallas as pl
from jax.experimental.pallas import tpu as pltpu
from jax.experimental.pallas import tpu_sc as plsc

N = 10000
E = 320000
D = 128

NC = 2
NS = 16
L = 16
NW = NC * NS

N_PAD = 10240
NTILE = N_PAD // NS
PADNODE = N_PAD - 1
C = 64
SR = 32
BLK_E = SR * C
NBLK = 160
E_PAD = NBLK * BLK_E
DEG_BPT = NBLK // NS
AGG_BPT = NBLK // NW
CH_AGG = AGG_BPT * SR
NBUF = 3


def _rsqrt16(x):
    i = lax.bitcast_convert_type(x, jnp.int32)
    i = jnp.int32(0x5F3759DF) - (i >> 1)
    y = lax.bitcast_convert_type(i, jnp.float32)
    for _ in range(3):
        y = y * (1.5 - 0.5 * x * y * y)
    return y


def _bcast16(ref, e):
    return plsc.load_gather(ref, [jnp.full((L,), 0, jnp.int32) + e])


def _sc_body(h_hbm, src_hbm, dst_hbm, srcA_hbm, dstA_hbm,
             z2d_hbm, z1d_hbm, ones_hbm,
             acc_out, tro_out, tri_out,
             sidx, didx, sidxA, rows0, rows1, rows2,
             roc, ric, coef_v, ones_v,
             hin_v, hout_v, idg_v, selfidx,
             gsem0, gsem1, gsem2, ssem0, ssem1, ssem2,
             csem0, csem1, csem2, dsem0, dsem1, dsem2,
             sh_ro, sh_ri, sh_acc):
    c = lax.axis_index("c")
    s = lax.axis_index("s")
    nbase = s * NTILE
    rows = (rows0, rows1, rows2)
    gsems = (gsem0, gsem1, gsem2)
    ssems = (ssem0, ssem1, ssem2)
    csems = (csem0, csem1, csem2)
    dsems = (dsem0, dsem1, dsem2)

    pltpu.sync_copy(z1d_hbm, sh_ri.at[pl.ds(nbase, NTILE)])
    pltpu.sync_copy(z1d_hbm, sh_ro.at[pl.ds(nbase, NTILE)])
    pltpu.sync_copy(z2d_hbm, sh_acc.at[pl.ds(nbase, NTILE)])
    pltpu.sync_copy(ones_hbm, ones_v)
    plsc.subcore_barrier()

    def deg_slab(kb, _):
        blk = s * DEG_BPT + kb
        pltpu.sync_copy(srcA_hbm.at[blk], sidxA)

        def srow_o(j, _):
            pltpu.sync_copy(ones_v, sh_ro.at[sidxA.at[j]], add=True)
            return 0
        lax.fori_loop(0, BLK_E // 128, srow_o, 0)
        pltpu.sync_copy(dstA_hbm.at[blk], sidxA)

        def srow_i(j, _):
            pltpu.sync_copy(ones_v, sh_ri.at[sidxA.at[j]], add=True)
            return 0
        lax.fori_loop(0, BLK_E // 128, srow_i, 0)
        return 0
    lax.fori_loop(0, DEG_BPT, deg_slab, 0)
    plsc.subcore_barrier()

    pltpu.sync_copy(sh_ri.at[pl.ds(nbase, NTILE)], hin_v)
    pltpu.sync_copy(sh_ro.at[pl.ds(nbase, NTILE)], hout_v)

    def tab_body(i, _):
        sl = pl.ds(i * L, L)
        cnt = hin_v[sl]
        deg_in = cnt + 1.0
        deg_out = hout_v[sl] + 1.0
        del cnt
        hin_v[sl] = _rsqrt16(deg_in)
        hout_v[sl] = _rsqrt16(deg_out)
        idg_v[sl] = 1.0 / deg_in
        return 0
    lax.fori_loop(0, NTILE // L, tab_body, 0)
    pltpu.sync_copy(hin_v, tri_out.at[pl.ds(c * N_PAD + nbase, NTILE)])
    pltpu.sync_copy(hout_v, tro_out.at[pl.ds(c * N_PAD + nbase, NTILE)])
    plsc.subcore_barrier()

    tid = c * NS + s
    blk0 = tid * AGG_BPT

    def stage_slab(kn):
        q = lax.div(kn, SR)
        bb = lax.rem(q, 2)

        @pl.when(lax.rem(kn, SR) == 0)
        def _():
            pltpu.sync_copy(src_hbm.at[blk0 + q], sidx.at[bb])
            pltpu.sync_copy(dst_hbm.at[blk0 + q], didx.at[bb])

    tro_c = tro_out.at[pl.ds(c * N_PAD, N_PAD)]
    tri_c = tri_out.at[pl.ds(c * N_PAD, N_PAD)]

    def issue_gather(kn, b):
        bb = lax.rem(lax.div(kn, SR), 2)
        jn = lax.rem(kn, SR)
        pltpu.async_copy(h_hbm.at[sidx.at[bb, jn]], rows[b], gsems[b])
        pltpu.async_copy(tro_c.at[sidx.at[bb, jn]],
                         roc.at[pl.ds(b * C, C)], csems[b])
        pltpu.async_copy(tri_c.at[didx.at[bb, jn]],
                         ric.at[pl.ds(b * C, C)], dsems[b])

    def wait_scatter(k, b):
        bb = lax.rem(lax.div(k, SR), 2)
        j = lax.rem(k, SR)
        pltpu.make_async_copy(rows[b], sh_acc.at[didx.at[bb, j]],
                              ssems[b]).wait()

    def process_chunk(k, b, steady):
        bb = lax.rem(lax.div(k, SR), 2)
        j = lax.rem(k, SR)
        pltpu.sync_copy(h_hbm.at[sidx.at[bb, j]], rows[b])
        pltpu.sync_copy(tro_c.at[sidx.at[bb, j]], roc.at[pl.ds(b * C, C)])
        pltpu.sync_copy(tri_c.at[didx.at[bb, j]], ric.at[pl.ds(b * C, C)])

        def coef_body(i, _):
            sl = pl.ds(b * C + i * L, L)
            ri = ric[sl]
            deg = 1.0 / (ri * ri)
            ic = 1.0 / jnp.maximum(deg - 1.0, 1.0)
            coef_v[sl] = roc[sl] * ri + ic
            return 0
        lax.fori_loop(0, C // L, coef_body, 0)

        def scale_body(e, _):
            cb = _bcast16(coef_v, b * C + e)
            for jj in range(D // L):
                sl = pl.ds(jj * L, L)
                rows[b][e, sl] = rows[b][e, sl] * cb
            return 0
        lax.fori_loop(0, C, scale_body, 0)

        kn = k + 2
        b2 = (b + 2) % 3
        if steady:
            @pl.when(kn < CH_AGG)
            def _():
                stage_slab(kn)

        pltpu.sync_copy(rows[b], sh_acc.at[didx.at[bb, j]], add=True)

    stage_slab(0)

    def triple(i, _):
        for u in range(3):
            process_chunk(i * 3 + u, u, True)
        return 0
    lax.fori_loop(0, (CH_AGG - 1) // 3, triple, 0)
    process_chunk(CH_AGG - 1, 0, False)

    @pl.when(c == 0)
    def _self_term():
        def self_body(kk, _):
            nb = nbase + kk * C
            pltpu.sync_copy(h_hbm.at[pl.ds(nb, C)], rows0)

            def idx_body(i, _):
                selfidx[pl.ds(i * L, L)] = lax.iota(jnp.int32, L) + nb + i * L
                return 0
            lax.fori_loop(0, C // L, idx_body, 0)

            def sscale_body(e, _):
                cb = _bcast16(idg_v, kk * C + e)
                for jj in range(D // L):
                    sl = pl.ds(jj * L, L)
                    rows0[e, sl] = rows0[e, sl] * cb
                return 0
            lax.fori_loop(0, C, sscale_body, 0)
            pltpu.sync_copy(rows0, sh_acc.at[selfidx], add=True)
            return 0
        lax.fori_loop(0, NTILE // C, self_body, 0)

    plsc.subcore_barrier()

    pltpu.sync_copy(sh_acc.at[pl.ds(nbase, NTILE)],
                    acc_out.at[c, pl.ds(nbase, NTILE)])


def _sc_aggregate(h, src3d, dst3d, srcA, dstA, z2d, z1d, ones):
    mesh = plsc.VectorSubcoreMesh(core_axis_name="c", subcore_axis_name="s")
    f = pl.kernel(
        _sc_body,
        out_type=(
            jax.ShapeDtypeStruct((NC, N_PAD, D), jnp.float32),
            jax.ShapeDtypeStruct((NC * N_PAD,), jnp.float32),
            jax.ShapeDtypeStruct((NC * N_PAD,), jnp.float32),
        ),
        mesh=mesh,
        scratch_types=[
            pltpu.VMEM((2, SR, C), jnp.int32),
            pltpu.VMEM((2, SR, C), jnp.int32),
            pltpu.VMEM((BLK_E // 128, 128), jnp.int32),
            pltpu.VMEM((C, D), jnp.float32),
            pltpu.VMEM((C, D), jnp.float32),
            pltpu.VMEM((C, D), jnp.float32),
            pltpu.VMEM((NBUF * C,), jnp.float32),
            pltpu.VMEM((NBUF * C,), jnp.float32),
            pltpu.VMEM((NBUF * C,), jnp.float32),
            pltpu.VMEM((128,), jnp.float32),
            pltpu.VMEM((NTILE,), jnp.float32),
            pltpu.VMEM((NTILE,), jnp.float32),
            pltpu.VMEM((NTILE,), jnp.float32),
            pltpu.VMEM((C,), jnp.int32),
            pltpu.SemaphoreType.DMA,
            pltpu.SemaphoreType.DMA,
            pltpu.SemaphoreType.DMA,
            pltpu.SemaphoreType.DMA,
            pltpu.SemaphoreType.DMA,
            pltpu.SemaphoreType.DMA,
            pltpu.SemaphoreType.DMA,
            pltpu.SemaphoreType.DMA,
            pltpu.SemaphoreType.DMA,
            pltpu.SemaphoreType.DMA,
            pltpu.SemaphoreType.DMA,
            pltpu.SemaphoreType.DMA,
            pltpu.VMEM_SHARED((N_PAD,), jnp.float32),
            pltpu.VMEM_SHARED((N_PAD,), jnp.float32),
            pltpu.VMEM_SHARED((N_PAD, D), jnp.float32),
        ],
        compiler_params=pltpu.CompilerParams(needs_layout_passes=False),
        name="homconv_sc_aggregate",
    )
    return f(h, src3d, dst3d, srcA, dstA, z2d, z1d, ones)


_BLKM = 2048
_BLKF = 2000


def _mm_body(x_ref, w_ref, b_ref, o_ref):
    o_ref[...] = lax.dot_general(
        x_ref[...], w_ref[...], (((1,), (1,)), ((), ())),
        preferred_element_type=jnp.float32) + b_ref[...]


def _matmul(x, w, b2d):
    return pl.pallas_call(
        _mm_body,
        grid=(N_PAD // _BLKM,),
        in_specs=[
            pl.BlockSpec((_BLKM, D), lambda i: (i, 0)),
            pl.BlockSpec((D, D), lambda i: (0, 0)),
            pl.BlockSpec((1, D), lambda i: (0, 0)),
        ],
        out_specs=pl.BlockSpec((_BLKM, D), lambda i: (i, 0)),
        out_shape=jax.ShapeDtypeStruct((N_PAD, D), jnp.float32),
    )(x, w, b2d)


def _fin_body(a_ref, o_ref):
    o_ref[...] = jnp.maximum(0.5 * (a_ref[0] + a_ref[1]), 0.0)


def _final(acc):
    return pl.pallas_call(
        _fin_body,
        grid=(N // _BLKF,),
        in_specs=[pl.BlockSpec((NC, _BLKF, D), lambda i: (0, i, 0))],
        out_specs=pl.BlockSpec((_BLKF, D), lambda i: (i, 0)),
        out_shape=jax.ShapeDtypeStruct((N, D), jnp.float32),
    )(acc)


def kernel(X, edge_index, W, b):
    x_pad = jnp.zeros((N_PAD, D), jnp.float32).at[:N].set(X)
    h = _matmul(x_pad, W, b.reshape(1, D))
    pad = jnp.full((E_PAD - E,), PADNODE, jnp.int32)
    src_flat = jnp.concatenate([edge_index[0], pad])
    dst_flat = jnp.concatenate([edge_index[1], pad])
    src3d = src_flat.reshape(NBLK, SR, C)
    dst3d = dst_flat.reshape(NBLK, SR, C)
    srcA = src_flat.reshape(NBLK, BLK_E // 128, 128)
    dstA = dst_flat.reshape(NBLK, BLK_E // 128, 128)
    z2d = jnp.zeros((NTILE, D), jnp.float32)
    z1d = jnp.zeros((NTILE,), jnp.float32)
    ones = jnp.ones((128,), jnp.float32)
    acc, _, _ = _sc_aggregate(h, src3d, dst3d, srcA, dstA, z2d, z1d, ones)
    return _final(acc)
